# baseline (speedup 1.0000x reference)
import functools

import jax
import jax.numpy as jnp
from jax import lax
from jax.experimental import pallas as pl
from jax.experimental.pallas import tpu as pltpu
from jax.experimental.pallas import tpu_sc as plsc

N = 10000
D = 128
E = 320000

NC = 2
NS = 16
NW = NC * NS
CH = 80
G = 16
NG = 8
EPW = NG * G * CH
E_PAD = NW * EPW
N_ACC = N + NW
RPS = 624
ZTAIL = N_ACC - NS * RPS
OTAIL = N - NS * RPS


def _mm_body(x_ref, wt_ref, z_ref):
    z_ref[...] = jnp.dot(x_ref[...], wt_ref[...],
                         preferred_element_type=jnp.float32)


def _matmul(x, w_t):
    return pl.pallas_call(
        _mm_body,
        grid=(10,),
        in_specs=[
            pl.BlockSpec((N // 10, D), lambda i: (i, 0)),
            pl.BlockSpec((D, D), lambda i: (0, 0)),
        ],
        out_specs=pl.BlockSpec((N // 10, D), lambda i: (i, 0)),
        out_shape=jax.ShapeDtypeStruct((N, D), jnp.float32),
    )(x, w_t)


_MESH = plsc.VectorSubcoreMesh(core_axis_name="c", subcore_axis_name="s")


@functools.partial(
    pl.kernel,
    out_type=jax.ShapeDtypeStruct((NC, N, D), jnp.float32),
    mesh=_MESH,
    scratch_types=[
        pltpu.VMEM((G, CH), jnp.int32),
        pltpu.VMEM((G, CH), jnp.int32),
        pltpu.VMEM((CH, D), jnp.float32),
        pltpu.VMEM((CH, D), jnp.float32),
        pltpu.VMEM_SHARED((N_ACC, D), jnp.float32),
        pltpu.SemaphoreType.DMA,
        pltpu.SemaphoreType.DMA,
        pltpu.SemaphoreType.DMA,
        pltpu.SemaphoreType.DMA,
    ],
)
def _sc_scatter(z_hbm, row_hbm, col_hbm, zeros_hbm, out_hbm,
                col_v, row_v, buf0, buf1, acc_sh, gs0, gs1, ss0, ss1):
    c = lax.axis_index("c")
    s = lax.axis_index("s")
    wid = s * NC + c

    stripe = pl.ds(pl.multiple_of(s * RPS, 8), RPS)
    pltpu.sync_copy(zeros_hbm.at[stripe], acc_sh.at[stripe])

    @pl.when(s == 0)
    def _zero_tail():
        ztail = pl.ds(NS * RPS, ZTAIL)
        pltpu.sync_copy(zeros_hbm.at[ztail], acc_sh.at[ztail])

    plsc.subcore_barrier()

    for g in range(NG):
        pltpu.sync_copy(col_hbm.at[wid].at[g], col_v)
        pltpu.sync_copy(row_hbm.at[wid].at[g], row_v)
        pltpu.async_copy(z_hbm.at[col_v.at[0]], buf0, gs0)

        def _pipe(jj, _):
            @pl.when(jj > 0)
            def _w0():
                pltpu.make_async_copy(buf1, acc_sh.at[row_v.at[jj - 1]],
                                      ss1).wait()

            pltpu.async_copy(z_hbm.at[col_v.at[jj + 1]], buf1, gs1)
            pltpu.make_async_copy(z_hbm.at[col_v.at[jj]], buf0, gs0).wait()
            pltpu.async_copy(buf0, acc_sh.at[row_v.at[jj]], ss0, add=True)
            pltpu.make_async_copy(buf0, acc_sh.at[row_v.at[jj]], ss0).wait()
            pltpu.async_copy(z_hbm.at[col_v.at[jj + 2]], buf0, gs0)
            pltpu.make_async_copy(z_hbm.at[col_v.at[jj + 1]], buf1,
                                  gs1).wait()
            pltpu.async_copy(buf1, acc_sh.at[row_v.at[jj + 1]], ss1,
                             add=True)
            return 0

        lax.fori_loop(0, (G - 2) // 2, lambda i, cy: _pipe(2 * i, cy), 0)

        pltpu.make_async_copy(buf1, acc_sh.at[row_v.at[G - 3]], ss1).wait()
        pltpu.async_copy(z_hbm.at[col_v.at[G - 1]], buf1, gs1)
        pltpu.make_async_copy(z_hbm.at[col_v.at[G - 2]], buf0, gs0).wait()
        pltpu.async_copy(buf0, acc_sh.at[row_v.at[G - 2]], ss0, add=True)
        pltpu.make_async_copy(z_hbm.at[col_v.at[G - 1]], buf1, gs1).wait()
        pltpu.async_copy(buf1, acc_sh.at[row_v.at[G - 1]], ss1, add=True)
        pltpu.make_async_copy(buf0, acc_sh.at[row_v.at[G - 2]], ss0).wait()
        pltpu.make_async_copy(buf1, acc_sh.at[row_v.at[G - 1]], ss1).wait()

    plsc.subcore_barrier()

    stripe_out = pl.ds(pl.multiple_of(s * RPS, 8), RPS)
    pltpu.sync_copy(acc_sh.at[stripe_out], out_hbm.at[c].at[stripe_out])

    @pl.when(s == 0)
    def _copy_tail():
        otail = pl.ds(NS * RPS, OTAIL)
        pltpu.sync_copy(acc_sh.at[otail], out_hbm.at[c].at[otail])


def _add_body(p_ref, o_ref):
    o_ref[...] = p_ref[0] + p_ref[1]


def _combine(partials):
    return pl.pallas_call(
        _add_body,
        grid=(10,),
        in_specs=[pl.BlockSpec((NC, N // 10, D), lambda i: (0, i, 0))],
        out_specs=pl.BlockSpec((N // 10, D), lambda i: (i, 0)),
        out_shape=jax.ShapeDtypeStruct((N, D), jnp.float32),
    )(partials)


def kernel(x, edge_index, W_fc, W_attn):
    z = _matmul(x, W_fc.T)
    ppw = EPW - E // NW
    dummy = (N + jnp.arange(NW, dtype=jnp.int32))[:, None] * jnp.ones(
        (1, ppw), jnp.int32)
    row = jnp.concatenate(
        [edge_index[0].reshape(NW, E // NW), dummy],
        axis=1).reshape(NW, NG, G, CH)
    col = jnp.concatenate(
        [edge_index[1].reshape(NW, E // NW),
         jnp.zeros((NW, ppw), jnp.int32)], axis=1).reshape(NW, NG, G, CH)
    zeros = jnp.zeros((N_ACC, D), dtype=jnp.float32)
    partials = _sc_scatter(z, row, col, zeros)
    return _combine(partials)


# speedup vs baseline: 2.5937x; 2.5937x over previous
import functools

import jax
import jax.numpy as jnp
from jax import lax
from jax.experimental import pallas as pl
from jax.experimental.pallas import tpu as pltpu
from jax.experimental.pallas import tpu_sc as plsc

N = 10000
D = 128
E = 320000

NC = 2
NS = 16
NW = NC * NS
CH = 80
G = 25
NG = 5
EPW = NG * G * CH
N_ACC = N
RPS = 624
ZTAIL = N_ACC - NS * RPS
OTAIL = N - NS * RPS


def _mm_body(x_ref, wt_ref, z_ref):
    z_ref[...] = jnp.dot(x_ref[...], wt_ref[...],
                         preferred_element_type=jnp.float32)


def _matmul(x, w_t):
    return pl.pallas_call(
        _mm_body,
        grid=(10,),
        in_specs=[
            pl.BlockSpec((N // 10, D), lambda i: (i, 0)),
            pl.BlockSpec((D, D), lambda i: (0, 0)),
        ],
        out_specs=pl.BlockSpec((N // 10, D), lambda i: (i, 0)),
        out_shape=jax.ShapeDtypeStruct((N, D), jnp.float32),
    )(x, w_t)


_MESH = plsc.VectorSubcoreMesh(core_axis_name="c", subcore_axis_name="s")


@functools.partial(
    pl.kernel,
    out_type=jax.ShapeDtypeStruct((NC, N, D), jnp.float32),
    mesh=_MESH,
    scratch_types=[
        pltpu.VMEM((G, CH), jnp.int32),
        pltpu.VMEM((G, CH), jnp.int32),
        pltpu.VMEM((CH, D), jnp.float32),
        pltpu.VMEM((CH, D), jnp.float32),
        pltpu.VMEM_SHARED((N_ACC, D), jnp.float32),
        pltpu.SemaphoreType.DMA,
        pltpu.SemaphoreType.DMA,
        pltpu.SemaphoreType.DMA,
        pltpu.SemaphoreType.DMA,
    ],
)
def _sc_scatter(z_hbm, row_hbm, col_hbm, zeros_hbm, out_hbm,
                col_v, row_v, buf0, buf1, acc_sh, gs0, gs1, ss0, ss1):
    c = lax.axis_index("c")
    s = lax.axis_index("s")
    wid = s * NC + c

    stripe = pl.ds(pl.multiple_of(s * RPS, 8), RPS)
    pltpu.sync_copy(zeros_hbm.at[stripe], acc_sh.at[stripe])

    @pl.when(s == 0)
    def _zero_tail():
        ztail = pl.ds(NS * RPS, ZTAIL)
        pltpu.sync_copy(zeros_hbm.at[ztail], acc_sh.at[ztail])

    plsc.subcore_barrier()

    for g in range(NG):
        pltpu.sync_copy(col_hbm.at[wid].at[g], col_v)
        pltpu.sync_copy(row_hbm.at[wid].at[g], row_v)
        pltpu.async_copy(z_hbm.at[col_v.at[0]], buf0, gs0)

        def _pipe(jj, _):
            @pl.when(jj > 0)
            def _w0():
                pltpu.make_async_copy(buf1, acc_sh.at[row_v.at[jj - 1]],
                                      ss1).wait()

            pltpu.async_copy(z_hbm.at[col_v.at[jj + 1]], buf1, gs1)
            pltpu.make_async_copy(z_hbm.at[col_v.at[jj]], buf0, gs0).wait()
            pltpu.async_copy(buf0, acc_sh.at[row_v.at[jj]], ss0, add=True)
            pltpu.make_async_copy(buf0, acc_sh.at[row_v.at[jj]], ss0).wait()
            pltpu.async_copy(z_hbm.at[col_v.at[jj + 2]], buf0, gs0)
            pltpu.make_async_copy(z_hbm.at[col_v.at[jj + 1]], buf1,
                                  gs1).wait()
            pltpu.async_copy(buf1, acc_sh.at[row_v.at[jj + 1]], ss1,
                             add=True)
            return 0

        lax.fori_loop(0, (G - 1) // 2, lambda i, cy: _pipe(2 * i, cy), 0)

        pltpu.make_async_copy(buf1, acc_sh.at[row_v.at[G - 2]], ss1).wait()
        pltpu.make_async_copy(z_hbm.at[col_v.at[G - 1]], buf0, gs0).wait()
        pltpu.async_copy(buf0, acc_sh.at[row_v.at[G - 1]], ss0, add=True)
        pltpu.make_async_copy(buf0, acc_sh.at[row_v.at[G - 1]], ss0).wait()

    plsc.subcore_barrier()

    stripe_out = pl.ds(pl.multiple_of(s * RPS, 8), RPS)
    pltpu.sync_copy(acc_sh.at[stripe_out], out_hbm.at[c].at[stripe_out])

    @pl.when(s == 0)
    def _copy_tail():
        otail = pl.ds(NS * RPS, OTAIL)
        pltpu.sync_copy(acc_sh.at[otail], out_hbm.at[c].at[otail])


def _add_body(p_ref, o_ref):
    o_ref[...] = p_ref[0] + p_ref[1]


def _combine(partials):
    return pl.pallas_call(
        _add_body,
        grid=(10,),
        in_specs=[pl.BlockSpec((NC, N // 10, D), lambda i: (0, i, 0))],
        out_specs=pl.BlockSpec((N // 10, D), lambda i: (i, 0)),
        out_shape=jax.ShapeDtypeStruct((N, D), jnp.float32),
    )(partials)


def kernel(x, edge_index, W_fc, W_attn):
    z = _matmul(x, W_fc.T)
    row = edge_index[0].reshape(NW, NG, G, CH)
    col = edge_index[1].reshape(NW, NG, G, CH)
    zeros = jnp.zeros((N_ACC, D), dtype=jnp.float32)
    partials = _sc_scatter(z, row, col, zeros)
    return _combine(partials)


# speedup vs baseline: 2.6068x; 1.0051x over previous
import functools

import jax
import jax.numpy as jnp
from jax import lax
from jax.experimental import pallas as pl
from jax.experimental.pallas import tpu as pltpu
from jax.experimental.pallas import tpu_sc as plsc

N = 10000
D = 128
E = 320000

NC = 2
NS = 16
NW = NC * NS
CH = 112
G = 15
NG = 6
EPW = NG * G * CH
PPW = EPW - E // NW
N_ACC = N + PPW
RPS = 624
ZTAIL = N_ACC - NS * RPS
OTAIL = N - NS * RPS


def _mm_body(x_ref, wt_ref, z_ref):
    z_ref[...] = jnp.dot(x_ref[...], wt_ref[...],
                         preferred_element_type=jnp.float32)


def _matmul(x, w_t):
    return pl.pallas_call(
        _mm_body,
        grid=(10,),
        in_specs=[
            pl.BlockSpec((N // 10, D), lambda i: (i, 0)),
            pl.BlockSpec((D, D), lambda i: (0, 0)),
        ],
        out_specs=pl.BlockSpec((N // 10, D), lambda i: (i, 0)),
        out_shape=jax.ShapeDtypeStruct((N, D), jnp.float32),
    )(x, w_t)


_MESH = plsc.VectorSubcoreMesh(core_axis_name="c", subcore_axis_name="s")


@functools.partial(
    pl.kernel,
    out_type=jax.ShapeDtypeStruct((NC, N, D), jnp.float32),
    mesh=_MESH,
    scratch_types=[
        pltpu.VMEM((G, CH), jnp.int32),
        pltpu.VMEM((G, CH), jnp.int32),
        pltpu.VMEM((CH, D), jnp.float32),
        pltpu.VMEM((CH, D), jnp.float32),
        pltpu.VMEM_SHARED((N_ACC, D), jnp.float32),
        pltpu.SemaphoreType.DMA,
        pltpu.SemaphoreType.DMA,
        pltpu.SemaphoreType.DMA,
        pltpu.SemaphoreType.DMA,
    ],
)
def _sc_scatter(z_hbm, row_hbm, col_hbm, zeros_hbm, out_hbm,
                col_v, row_v, buf0, buf1, acc_sh, gs0, gs1, ss0, ss1):
    c = lax.axis_index("c")
    s = lax.axis_index("s")
    wid = s * NC + c

    stripe = pl.ds(pl.multiple_of(s * RPS, 8), RPS)
    pltpu.sync_copy(zeros_hbm.at[stripe], acc_sh.at[stripe])

    @pl.when(s == 0)
    def _zero_tail():
        ztail = pl.ds(NS * RPS, ZTAIL)
        pltpu.sync_copy(zeros_hbm.at[ztail], acc_sh.at[ztail])

    plsc.subcore_barrier()

    for g in range(NG):
        pltpu.sync_copy(col_hbm.at[wid].at[g], col_v)
        pltpu.sync_copy(row_hbm.at[wid].at[g], row_v)
        pltpu.async_copy(z_hbm.at[col_v.at[0]], buf0, gs0)

        def _pipe(jj, _):
            @pl.when(jj > 0)
            def _w0():
                pltpu.make_async_copy(buf1, acc_sh.at[row_v.at[jj - 1]],
                                      ss1).wait()

            pltpu.async_copy(z_hbm.at[col_v.at[jj + 1]], buf1, gs1)
            pltpu.make_async_copy(z_hbm.at[col_v.at[jj]], buf0, gs0).wait()
            pltpu.async_copy(buf0, acc_sh.at[row_v.at[jj]], ss0, add=True)
            pltpu.make_async_copy(buf0, acc_sh.at[row_v.at[jj]], ss0).wait()
            pltpu.async_copy(z_hbm.at[col_v.at[jj + 2]], buf0, gs0)
            pltpu.make_async_copy(z_hbm.at[col_v.at[jj + 1]], buf1,
                                  gs1).wait()
            pltpu.async_copy(buf1, acc_sh.at[row_v.at[jj + 1]], ss1,
                             add=True)
            return 0

        lax.fori_loop(0, (G - 1) // 2, lambda i, cy: _pipe(2 * i, cy), 0)

        pltpu.make_async_copy(buf1, acc_sh.at[row_v.at[G - 2]], ss1).wait()
        pltpu.make_async_copy(z_hbm.at[col_v.at[G - 1]], buf0, gs0).wait()
        pltpu.async_copy(buf0, acc_sh.at[row_v.at[G - 1]], ss0, add=True)
        pltpu.make_async_copy(buf0, acc_sh.at[row_v.at[G - 1]], ss0).wait()

    plsc.subcore_barrier()

    stripe_out = pl.ds(pl.multiple_of(s * RPS, 8), RPS)
    pltpu.sync_copy(acc_sh.at[stripe_out], out_hbm.at[c].at[stripe_out])

    @pl.when(s == 0)
    def _copy_tail():
        otail = pl.ds(NS * RPS, OTAIL)
        pltpu.sync_copy(acc_sh.at[otail], out_hbm.at[c].at[otail])


def _add_body(p_ref, o_ref):
    o_ref[...] = p_ref[0] + p_ref[1]


def _combine(partials):
    return pl.pallas_call(
        _add_body,
        grid=(10,),
        in_specs=[pl.BlockSpec((NC, N // 10, D), lambda i: (0, i, 0))],
        out_specs=pl.BlockSpec((N // 10, D), lambda i: (i, 0)),
        out_shape=jax.ShapeDtypeStruct((N, D), jnp.float32),
    )(partials)


def kernel(x, edge_index, W_fc, W_attn):
    z = _matmul(x, W_fc.T)
    dummy_r = jnp.broadcast_to(N + jnp.arange(PPW, dtype=jnp.int32),
                               (NW, PPW))
    dummy_c = jnp.broadcast_to(jnp.arange(PPW, dtype=jnp.int32), (NW, PPW))
    row = jnp.concatenate(
        [edge_index[0].reshape(NW, E // NW), dummy_r],
        axis=1).reshape(NW, NG, G, CH)
    col = jnp.concatenate(
        [edge_index[1].reshape(NW, E // NW), dummy_c],
        axis=1).reshape(NW, NG, G, CH)
    zeros = jnp.zeros((N_ACC, D), dtype=jnp.float32)
    partials = _sc_scatter(z, row, col, zeros)
    return _combine(partials)


# speedup vs baseline: 2.7898x; 1.0702x over previous
import functools

import jax
import jax.numpy as jnp
from jax import lax
from jax.experimental import pallas as pl
from jax.experimental.pallas import tpu as pltpu
from jax.experimental.pallas import tpu_sc as plsc

N = 10000
D = 128
E = 320000

NC = 2
NS = 16
NW = NC * NS
CH = 112
G = 45
NG = 2
EPW = NG * G * CH
PPW = EPW - E // NW
N_ACC = N + PPW
RPS = 624
ZTAIL = N_ACC - NS * RPS
OTAIL = N - NS * RPS


def _mm_body(x_ref, wt_ref, z_ref):
    z_ref[...] = jnp.dot(x_ref[...], wt_ref[...],
                         preferred_element_type=jnp.float32)


def _matmul(x, w_t):
    return pl.pallas_call(
        _mm_body,
        grid=(10,),
        in_specs=[
            pl.BlockSpec((N // 10, D), lambda i: (i, 0)),
            pl.BlockSpec((D, D), lambda i: (0, 0)),
        ],
        out_specs=pl.BlockSpec((N // 10, D), lambda i: (i, 0)),
        out_shape=jax.ShapeDtypeStruct((N, D), jnp.float32),
    )(x, w_t)


_MESH = plsc.VectorSubcoreMesh(core_axis_name="c", subcore_axis_name="s")


@functools.partial(
    pl.kernel,
    out_type=jax.ShapeDtypeStruct((NC, N, D), jnp.float32),
    mesh=_MESH,
    scratch_types=[
        pltpu.VMEM((G, CH), jnp.int32),
        pltpu.VMEM((G, CH), jnp.int32),
        pltpu.VMEM((CH, D), jnp.float32),
        pltpu.VMEM((CH, D), jnp.float32),
        pltpu.VMEM_SHARED((N_ACC, D), jnp.float32),
        pltpu.SemaphoreType.DMA,
        pltpu.SemaphoreType.DMA,
        pltpu.SemaphoreType.DMA,
        pltpu.SemaphoreType.DMA,
    ],
)
def _sc_scatter(z_hbm, row_hbm, col_hbm, zeros_hbm, out_hbm,
                col_v, row_v, buf0, buf1, acc_sh, gs0, gs1, ss0, ss1):
    c = lax.axis_index("c")
    s = lax.axis_index("s")
    wid = s * NC + c

    stripe = pl.ds(pl.multiple_of(s * RPS, 8), RPS)
    pltpu.sync_copy(zeros_hbm.at[stripe], acc_sh.at[stripe])

    @pl.when(s == 0)
    def _zero_tail():
        ztail = pl.ds(NS * RPS, ZTAIL)
        pltpu.sync_copy(zeros_hbm.at[ztail], acc_sh.at[ztail])

    plsc.subcore_barrier()

    for g in range(NG):
        pltpu.sync_copy(col_hbm.at[wid].at[g], col_v)
        pltpu.sync_copy(row_hbm.at[wid].at[g], row_v)
        pltpu.async_copy(z_hbm.at[col_v.at[0]], buf0, gs0)

        def _pipe(jj, _):
            @pl.when(jj > 0)
            def _w0():
                pltpu.make_async_copy(buf1, acc_sh.at[row_v.at[jj - 1]],
                                      ss1).wait()

            pltpu.async_copy(z_hbm.at[col_v.at[jj + 1]], buf1, gs1)
            pltpu.make_async_copy(z_hbm.at[col_v.at[jj]], buf0, gs0).wait()
            pltpu.async_copy(buf0, acc_sh.at[row_v.at[jj]], ss0, add=True)
            pltpu.make_async_copy(buf0, acc_sh.at[row_v.at[jj]], ss0).wait()
            pltpu.async_copy(z_hbm.at[col_v.at[jj + 2]], buf0, gs0)
            pltpu.make_async_copy(z_hbm.at[col_v.at[jj + 1]], buf1,
                                  gs1).wait()
            pltpu.async_copy(buf1, acc_sh.at[row_v.at[jj + 1]], ss1,
                             add=True)
            return 0

        lax.fori_loop(0, (G - 1) // 2, lambda i, cy: _pipe(2 * i, cy), 0)

        pltpu.make_async_copy(buf1, acc_sh.at[row_v.at[G - 2]], ss1).wait()
        pltpu.make_async_copy(z_hbm.at[col_v.at[G - 1]], buf0, gs0).wait()
        pltpu.async_copy(buf0, acc_sh.at[row_v.at[G - 1]], ss0, add=True)
        pltpu.make_async_copy(buf0, acc_sh.at[row_v.at[G - 1]], ss0).wait()

    plsc.subcore_barrier()

    stripe_out = pl.ds(pl.multiple_of(s * RPS, 8), RPS)
    pltpu.sync_copy(acc_sh.at[stripe_out], out_hbm.at[c].at[stripe_out])

    @pl.when(s == 0)
    def _copy_tail():
        otail = pl.ds(NS * RPS, OTAIL)
        pltpu.sync_copy(acc_sh.at[otail], out_hbm.at[c].at[otail])


def _add_body(p_ref, o_ref):
    o_ref[...] = p_ref[0] + p_ref[1]


def _combine(partials):
    return pl.pallas_call(
        _add_body,
        grid=(10,),
        in_specs=[pl.BlockSpec((NC, N // 10, D), lambda i: (0, i, 0))],
        out_specs=pl.BlockSpec((N // 10, D), lambda i: (i, 0)),
        out_shape=jax.ShapeDtypeStruct((N, D), jnp.float32),
    )(partials)


def kernel(x, edge_index, W_fc, W_attn):
    z = _matmul(x, W_fc.T)
    dummy_r = jnp.broadcast_to(N + jnp.arange(PPW, dtype=jnp.int32),
                               (NW, PPW))
    dummy_c = jnp.broadcast_to(jnp.arange(PPW, dtype=jnp.int32), (NW, PPW))
    row = jnp.concatenate(
        [edge_index[0].reshape(NW, E // NW), dummy_r],
        axis=1).reshape(NW, NG, G, CH)
    col = jnp.concatenate(
        [edge_index[1].reshape(NW, E // NW), dummy_c],
        axis=1).reshape(NW, NG, G, CH)
    zeros = jnp.zeros((N_ACC, D), dtype=jnp.float32)
    partials = _sc_scatter(z, row, col, zeros)
    return _combine(partials)


# speedup vs baseline: 2.8390x; 1.0176x over previous
import functools

import jax
import jax.numpy as jnp
from jax import lax
from jax.experimental import pallas as pl
from jax.experimental.pallas import tpu as pltpu
from jax.experimental.pallas import tpu_sc as plsc

N = 10000
D = 128
E = 320000

NC = 2
NS = 16
NW = NC * NS
CH = 112
G = 45
NG = 2
EPW = NG * G * CH
PPW = EPW - E // NW
N_ACC = N + PPW
RPS = 624
ZTAIL = N_ACC - NS * RPS
OTAIL = N - NS * RPS


def _mm_body(x_ref, w_ref, z_ref, zero_ref):
    z_ref[...] = lax.dot_general(x_ref[...], w_ref[...],
                                 (((1,), (1,)), ((), ())),
                                 preferred_element_type=jnp.float32)
    zero_ref[...] = jnp.zeros_like(zero_ref)


def _matmul(x, w):
    return pl.pallas_call(
        _mm_body,
        grid=(10,),
        in_specs=[
            pl.BlockSpec((N // 10, D), lambda i: (i, 0)),
            pl.BlockSpec((D, D), lambda i: (0, 0)),
        ],
        out_specs=[
            pl.BlockSpec((N // 10, D), lambda i: (i, 0)),
            pl.BlockSpec((N_ACC // 10, D), lambda i: (i, 0)),
        ],
        out_shape=[
            jax.ShapeDtypeStruct((N, D), jnp.float32),
            jax.ShapeDtypeStruct((N_ACC, D), jnp.float32),
        ],
    )(x, w)


_MESH = plsc.VectorSubcoreMesh(core_axis_name="c", subcore_axis_name="s")


@functools.partial(
    pl.kernel,
    out_type=jax.ShapeDtypeStruct((NC, N, D), jnp.float32),
    mesh=_MESH,
    scratch_types=[
        pltpu.VMEM((G, CH), jnp.int32),
        pltpu.VMEM((G, CH), jnp.int32),
        pltpu.VMEM((CH, D), jnp.float32),
        pltpu.VMEM((CH, D), jnp.float32),
        pltpu.VMEM_SHARED((N_ACC, D), jnp.float32),
        pltpu.SemaphoreType.DMA,
        pltpu.SemaphoreType.DMA,
        pltpu.SemaphoreType.DMA,
        pltpu.SemaphoreType.DMA,
    ],
)
def _sc_scatter(z_hbm, row_hbm, col_hbm, zeros_hbm, out_hbm,
                col_v, row_v, buf0, buf1, acc_sh, gs0, gs1, ss0, ss1):
    c = lax.axis_index("c")
    s = lax.axis_index("s")
    wid = s * NC + c

    stripe = pl.ds(pl.multiple_of(s * RPS, 8), RPS)
    pltpu.sync_copy(zeros_hbm.at[stripe], acc_sh.at[stripe])

    @pl.when(s == 0)
    def _zero_tail():
        ztail = pl.ds(NS * RPS, ZTAIL)
        pltpu.sync_copy(zeros_hbm.at[ztail], acc_sh.at[ztail])

    plsc.subcore_barrier()

    for g in range(NG):
        pltpu.sync_copy(col_hbm.at[wid].at[g], col_v)
        pltpu.sync_copy(row_hbm.at[wid].at[g], row_v)
        pltpu.async_copy(z_hbm.at[col_v.at[0]], buf0, gs0)

        def _pipe(jj, _):
            @pl.when(jj > 0)
            def _w0():
                pltpu.make_async_copy(buf1, acc_sh.at[row_v.at[jj - 1]],
                                      ss1).wait()

            pltpu.async_copy(z_hbm.at[col_v.at[jj + 1]], buf1, gs1)
            pltpu.make_async_copy(z_hbm.at[col_v.at[jj]], buf0, gs0).wait()
            pltpu.async_copy(buf0, acc_sh.at[row_v.at[jj]], ss0, add=True)
            pltpu.make_async_copy(buf0, acc_sh.at[row_v.at[jj]], ss0).wait()
            pltpu.async_copy(z_hbm.at[col_v.at[jj + 2]], buf0, gs0)
            pltpu.make_async_copy(z_hbm.at[col_v.at[jj + 1]], buf1,
                                  gs1).wait()
            pltpu.async_copy(buf1, acc_sh.at[row_v.at[jj + 1]], ss1,
                             add=True)
            return 0

        lax.fori_loop(0, (G - 1) // 2, lambda i, cy: _pipe(2 * i, cy), 0)

        pltpu.make_async_copy(buf1, acc_sh.at[row_v.at[G - 2]], ss1).wait()
        pltpu.make_async_copy(z_hbm.at[col_v.at[G - 1]], buf0, gs0).wait()
        pltpu.async_copy(buf0, acc_sh.at[row_v.at[G - 1]], ss0, add=True)
        pltpu.make_async_copy(buf0, acc_sh.at[row_v.at[G - 1]], ss0).wait()

    plsc.subcore_barrier()

    stripe_out = pl.ds(pl.multiple_of(s * RPS, 8), RPS)
    pltpu.sync_copy(acc_sh.at[stripe_out], out_hbm.at[c].at[stripe_out])

    @pl.when(s == 0)
    def _copy_tail():
        otail = pl.ds(NS * RPS, OTAIL)
        pltpu.sync_copy(acc_sh.at[otail], out_hbm.at[c].at[otail])


def _add_body(p_ref, o_ref):
    o_ref[...] = p_ref[0] + p_ref[1]


def _combine(partials):
    return pl.pallas_call(
        _add_body,
        grid=(10,),
        in_specs=[pl.BlockSpec((NC, N // 10, D), lambda i: (0, i, 0))],
        out_specs=pl.BlockSpec((N // 10, D), lambda i: (i, 0)),
        out_shape=jax.ShapeDtypeStruct((N, D), jnp.float32),
    )(partials)


def kernel(x, edge_index, W_fc, W_attn):
    z, zeros = _matmul(x, W_fc)
    dummy_r = jnp.broadcast_to(N + jnp.arange(PPW, dtype=jnp.int32),
                               (NW, PPW))
    dummy_c = jnp.broadcast_to(jnp.arange(PPW, dtype=jnp.int32), (NW, PPW))
    row = jnp.concatenate(
        [edge_index[0].reshape(NW, E // NW), dummy_r],
        axis=1).reshape(NW, NG, G, CH)
    col = jnp.concatenate(
        [edge_index[1].reshape(NW, E // NW), dummy_c],
        axis=1).reshape(NW, NG, G, CH)
    partials = _sc_scatter(z, row, col, zeros)
    return _combine(partials)


# speedup vs baseline: 3.1830x; 1.1212x over previous
import functools

import jax
import jax.numpy as jnp
from jax import lax
from jax.experimental import pallas as pl
from jax.experimental.pallas import tpu as pltpu
from jax.experimental.pallas import tpu_sc as plsc

N = 10000
D = 128
E = 320000

NC = 2
NS = 16
NW = NC * NS
CH = 112
G = 45
NG = 2
EPW = NG * G * CH
PPW = EPW - E // NW
NZ = N + PPW
RPS = 624
TAIL = N - NS * RPS


def _mm_body(x_ref, w_ref, z_ref, zero_ref):
    z_ref[:N, :] = lax.dot_general(x_ref[...], w_ref[...],
                                   (((1,), (1,)), ((), ())),
                                   preferred_element_type=jnp.float32)
    z_ref[N:, :] = jnp.zeros((NZ - N, D), jnp.float32)
    zero_ref[...] = jnp.zeros_like(zero_ref)


def _matmul(x, w):
    return pl.pallas_call(
        _mm_body,
        out_shape=[
            jax.ShapeDtypeStruct((NZ, D), jnp.float32),
            jax.ShapeDtypeStruct((N, D), jnp.float32),
        ],
    )(x, w)


_MESH = plsc.VectorSubcoreMesh(core_axis_name="c", subcore_axis_name="s")


@functools.partial(
    pl.kernel,
    out_type=jax.ShapeDtypeStruct((NC, N, D), jnp.float32),
    mesh=_MESH,
    scratch_types=[
        pltpu.VMEM((G, CH), jnp.int32),
        pltpu.VMEM((G, CH), jnp.int32),
        pltpu.VMEM((CH, D), jnp.float32),
        pltpu.VMEM((CH, D), jnp.float32),
        pltpu.VMEM_SHARED((N, D), jnp.float32),
        pltpu.SemaphoreType.DMA,
        pltpu.SemaphoreType.DMA,
        pltpu.SemaphoreType.DMA,
        pltpu.SemaphoreType.DMA,
    ],
)
def _sc_scatter(z_hbm, edges_hbm, zeros_hbm, out_hbm,
                col_v, row_v, buf0, buf1, acc_sh, gs0, gs1, ss0, ss1):
    c = lax.axis_index("c")
    s = lax.axis_index("s")
    wid = s * NC + c

    stripe = pl.ds(pl.multiple_of(s * RPS, 8), RPS)
    tail = pl.ds(NS * RPS, TAIL)
    pltpu.sync_copy(zeros_hbm.at[stripe], acc_sh.at[stripe])

    @pl.when(s == 0)
    def _zero_tail():
        pltpu.sync_copy(zeros_hbm.at[tail], acc_sh.at[tail])

    plsc.subcore_barrier()

    for g in range(NG):
        pltpu.sync_copy(edges_hbm.at[0].at[wid].at[g], row_v)
        pltpu.sync_copy(edges_hbm.at[1].at[wid].at[g], col_v)
        pltpu.async_copy(z_hbm.at[col_v.at[0]], buf0, gs0)

        def _pipe(jj, _):
            @pl.when(jj > 0)
            def _w0():
                pltpu.make_async_copy(buf1, acc_sh.at[row_v.at[jj - 1]],
                                      ss1).wait()

            pltpu.async_copy(z_hbm.at[col_v.at[jj + 1]], buf1, gs1)
            pltpu.make_async_copy(z_hbm.at[col_v.at[jj]], buf0, gs0).wait()
            pltpu.async_copy(buf0, acc_sh.at[row_v.at[jj]], ss0, add=True)
            pltpu.make_async_copy(buf0, acc_sh.at[row_v.at[jj]], ss0).wait()
            pltpu.async_copy(z_hbm.at[col_v.at[jj + 2]], buf0, gs0)
            pltpu.make_async_copy(z_hbm.at[col_v.at[jj + 1]], buf1,
                                  gs1).wait()
            pltpu.async_copy(buf1, acc_sh.at[row_v.at[jj + 1]], ss1,
                             add=True)
            return 0

        lax.fori_loop(0, (G - 1) // 2, lambda i, cy: _pipe(2 * i, cy), 0)

        pltpu.make_async_copy(buf1, acc_sh.at[row_v.at[G - 2]], ss1).wait()
        pltpu.make_async_copy(z_hbm.at[col_v.at[G - 1]], buf0, gs0).wait()
        pltpu.async_copy(buf0, acc_sh.at[row_v.at[G - 1]], ss0, add=True)
        pltpu.make_async_copy(buf0, acc_sh.at[row_v.at[G - 1]], ss0).wait()

    plsc.subcore_barrier()

    pltpu.sync_copy(acc_sh.at[stripe], out_hbm.at[c].at[stripe])

    @pl.when(s == 0)
    def _copy_tail():
        pltpu.sync_copy(acc_sh.at[tail], out_hbm.at[c].at[tail])


def _add_body(p_ref, o_ref):
    o_ref[...] = p_ref[0] + p_ref[1]


def _combine(partials):
    return pl.pallas_call(
        _add_body,
        out_shape=jax.ShapeDtypeStruct((N, D), jnp.float32),
    )(partials)


def kernel(x, edge_index, W_fc, W_attn):
    z_ext, zeros = _matmul(x, W_fc)
    tail_vals = jnp.stack([jnp.arange(PPW, dtype=jnp.int32),
                           N + jnp.arange(PPW, dtype=jnp.int32)])
    tails = jnp.broadcast_to(tail_vals[:, None, :], (2, NW, PPW))
    edges = jnp.concatenate(
        [edge_index.reshape(2, NW, E // NW), tails],
        axis=2).reshape(2, NW, NG, G, CH)
    partials = _sc_scatter(z_ext, edges, zeros)
    return _combine(partials)
